# initial kernel scaffold (unmeasured)
import jax
import jax.numpy as jnp
from jax import lax
from jax.experimental import pallas as pl
from jax.experimental.pallas import tpu as pltpu

N_DEV = 4


def kernel(x, w_mat):
    m_per, k = x.shape
    _, n = w_mat.shape
    n_per = n // N_DEV

    def body(x_ref, w_ref, out_ref, send_ref, comm_ref, send_sems, recv_sems):
        my = lax.axis_index("i")

        barrier_sem = pltpu.get_barrier_semaphore()
        for off in (1, 2, 3):
            pl.semaphore_signal(
                barrier_sem, inc=1,
                device_id=((my + off) % N_DEV,),
                device_id_type=pl.DeviceIdType.MESH,
            )
        pl.semaphore_wait(barrier_sem, 3)

        y = jnp.dot(x_ref[:, :], w_ref[:, :], preferred_element_type=jnp.float32)
        yb = y.astype(jnp.bfloat16)
        for j in range(N_DEV):
            send_ref[j] = yb[:, j * n_per:(j + 1) * n_per]

        out_ref[pl.ds(my * m_per, m_per), :] = lax.dynamic_slice(
            y, (0, my * n_per), (m_per, n_per))

        sends = []
        for off in (1, 2, 3):
            peer = (my + off) % N_DEV
            rdma = pltpu.make_async_remote_copy(
                src_ref=send_ref.at[peer],
                dst_ref=comm_ref.at[my],
                send_sem=send_sems.at[off - 1],
                recv_sem=recv_sems.at[off - 1],
                device_id=(peer,),
                device_id_type=pl.DeviceIdType.MESH,
            )
            rdma.start()
            sends.append(rdma)

        for o in (1, 2, 3):
            src = (my - o) % N_DEV
            recv = pltpu.make_async_remote_copy(
                src_ref=send_ref.at[src],
                dst_ref=comm_ref.at[src],
                send_sem=send_sems.at[o - 1],
                recv_sem=recv_sems.at[o - 1],
                device_id=(src,),
                device_id_type=pl.DeviceIdType.MESH,
            )
            recv.wait_recv()
            out_ref[pl.ds(src * m_per, m_per), :] = comm_ref[src].astype(
                jnp.float32)

        for rdma in sends:
            rdma.wait_send()

    return pl.pallas_call(
        body,
        out_shape=jax.ShapeDtypeStruct((N_DEV * m_per, n_per), jnp.float32),
        in_specs=[
            pl.BlockSpec(memory_space=pltpu.VMEM),
            pl.BlockSpec(memory_space=pltpu.VMEM),
        ],
        out_specs=pl.BlockSpec(memory_space=pltpu.VMEM),
        scratch_shapes=[
            pltpu.VMEM((N_DEV, m_per, n_per), jnp.bfloat16),
            pltpu.VMEM((N_DEV, m_per, n_per), jnp.bfloat16),
            pltpu.SemaphoreType.DMA((3,)),
            pltpu.SemaphoreType.DMA((3,)),
        ],
        compiler_params=pltpu.CompilerParams(collective_id=0),
    )(x, w_mat)


# baseline (device time: 12986 ns/iter reference)
import jax
import jax.numpy as jnp
from jax import lax
from jax.experimental import pallas as pl
from jax.experimental.pallas import tpu as pltpu

N_DEV = 4


def kernel(x, w_mat):
    m_per, k = x.shape
    _, n = w_mat.shape
    n_per = n // N_DEV

    def body(x_ref, w_ref, out_ref, send_ref, comm_ref, send_sems, recv_sems):
        my = lax.axis_index("i")

        barrier_sem = pltpu.get_barrier_semaphore()
        for off in (1, 2, 3):
            pl.semaphore_signal(
                barrier_sem, inc=1,
                device_id=((my + off) % N_DEV,),
                device_id_type=pl.DeviceIdType.MESH,
            )
        pl.semaphore_wait(barrier_sem, 3)

        y = jnp.dot(x_ref[:, :], w_ref[:, :], preferred_element_type=jnp.float32)
        yb = y.astype(jnp.bfloat16)
        for j in range(N_DEV):
            send_ref[j] = yb[:, j * n_per:(j + 1) * n_per]

        out_ref[pl.ds(my * m_per, m_per), :] = send_ref[my].astype(jnp.float32)

        sends = []
        for off in (1, 2, 3):
            peer = (my + off) % N_DEV
            rdma = pltpu.make_async_remote_copy(
                src_ref=send_ref.at[peer],
                dst_ref=comm_ref.at[my],
                send_sem=send_sems.at[off - 1],
                recv_sem=recv_sems.at[off - 1],
                device_id=(peer,),
                device_id_type=pl.DeviceIdType.MESH,
            )
            rdma.start()
            sends.append(rdma)

        for o in (1, 2, 3):
            src = (my - o) % N_DEV
            recv = pltpu.make_async_remote_copy(
                src_ref=send_ref.at[src],
                dst_ref=comm_ref.at[src],
                send_sem=send_sems.at[o - 1],
                recv_sem=recv_sems.at[o - 1],
                device_id=(src,),
                device_id_type=pl.DeviceIdType.MESH,
            )
            recv.wait_recv()
            out_ref[pl.ds(src * m_per, m_per), :] = comm_ref[src].astype(
                jnp.float32)

        for rdma in sends:
            rdma.wait_send()

    return pl.pallas_call(
        body,
        out_shape=jax.ShapeDtypeStruct((N_DEV * m_per, n_per), jnp.float32),
        in_specs=[
            pl.BlockSpec(memory_space=pltpu.VMEM),
            pl.BlockSpec(memory_space=pltpu.VMEM),
        ],
        out_specs=pl.BlockSpec(memory_space=pltpu.VMEM),
        scratch_shapes=[
            pltpu.VMEM((N_DEV, m_per, n_per), jnp.bfloat16),
            pltpu.VMEM((N_DEV, m_per, n_per), jnp.bfloat16),
            pltpu.SemaphoreType.DMA((3,)),
            pltpu.SemaphoreType.DMA((3,)),
        ],
        compiler_params=pltpu.CompilerParams(collective_id=0),
    )(x, w_mat)
